# baseline (speedup 1.0000x reference)
import jax
import jax.numpy as jnp
from jax import lax
from jax.experimental import pallas as pl
from jax.experimental.pallas import tpu as pltpu
from jax.experimental.pallas import tpu_sc as plsc

B = 16384
K = 32
NC, NS, L = 2, 16, 16
NW = NC * NS
CHUNK = B // NW
IDX_W = 128
IDX_ROWS = CHUNK // IDX_W
GROUPS = CHUNK // L
NUSERS = 1000000
NITEMS = 100000
GLOBAL_MEAN = 3.5


def _mf_body(uid_hbm, iid_hbm, w1_hbm, u1_hbm, bu_hbm, bi_hbm, out_hbm,
             uid_v, iid_v, wcols, ucols, bu_v, bi_v, out_v, sem):
    wid = lax.axis_index("s") * NC + lax.axis_index("c")
    row0 = wid * IDX_ROWS
    pltpu.sync_copy(uid_hbm.at[pl.ds(row0, IDX_ROWS)], uid_v)
    pltpu.sync_copy(iid_hbm.at[pl.ds(row0, IDX_ROWS)], iid_v)

    copies = []
    for j in range(IDX_ROWS):
        sl = pl.ds(j * IDX_W, IDX_W)
        copies.append(pltpu.async_copy(bu_hbm.at[uid_v.at[j]], bu_v.at[sl], sem))
        copies.append(pltpu.async_copy(bi_hbm.at[iid_v.at[j]], bi_v.at[sl], sem))
    for k in range(K):
        for j in range(IDX_ROWS):
            sl = pl.ds(j * IDX_W, IDX_W)
            copies.append(pltpu.async_copy(
                w1_hbm.at[pl.ds(k * NUSERS, NUSERS)].at[uid_v.at[j]],
                wcols.at[k, sl], sem))
            copies.append(pltpu.async_copy(
                u1_hbm.at[pl.ds(k * NITEMS, NITEMS)].at[iid_v.at[j]],
                ucols.at[k, sl], sem))
    for c in copies:
        c.wait()

    def group(g, carry):
        o = g * L
        acc = bu_v[pl.ds(o, L)] + bi_v[pl.ds(o, L)] + jnp.float32(GLOBAL_MEAN)
        for k in range(K):
            acc = acc + wcols[k, pl.ds(o, L)] * ucols[k, pl.ds(o, L)]
        out_v[pl.ds(o, L)] = acc
        return carry

    lax.fori_loop(0, GROUPS, group, 0)
    pltpu.sync_copy(out_v, out_hbm.at[pl.ds(wid * CHUNK, CHUNK)])


def kernel(user_ids, item_ids, W, U, bias_user, bias_item):
    uid2 = user_ids.astype(jnp.int32).reshape(NW * IDX_ROWS, IDX_W)
    iid2 = item_ids.astype(jnp.int32).reshape(NW * IDX_ROWS, IDX_W)
    w1 = W.T.reshape(-1)
    u1 = U.T.reshape(-1)
    mesh = plsc.VectorSubcoreMesh(core_axis_name="c", subcore_axis_name="s",
                                  num_cores=NC, num_subcores=NS)
    f = pl.kernel(
        _mf_body,
        out_type=jax.ShapeDtypeStruct((B,), jnp.float32),
        mesh=mesh,
        compiler_params=pltpu.CompilerParams(needs_layout_passes=False,
                                             use_tc_tiling_on_sc=False),
        scratch_types=[
            pltpu.VMEM((IDX_ROWS, IDX_W), jnp.int32),
            pltpu.VMEM((IDX_ROWS, IDX_W), jnp.int32),
            pltpu.VMEM((K, CHUNK), jnp.float32),
            pltpu.VMEM((K, CHUNK), jnp.float32),
            pltpu.VMEM((CHUNK,), jnp.float32),
            pltpu.VMEM((CHUNK,), jnp.float32),
            pltpu.VMEM((CHUNK,), jnp.float32),
            pltpu.SemaphoreType.DMA,
        ],
    )
    return f(uid2, iid2, w1, u1, bias_user, bias_item)


# speedup vs baseline: 4.6772x; 4.6772x over previous
import jax
import jax.numpy as jnp
from jax import lax
from jax.experimental import pallas as pl
from jax.experimental.pallas import tpu as pltpu
from jax.experimental.pallas import tpu_sc as plsc

B = 16384
K = 32
FOLD = 128 // K
NC, NS, L = 2, 16, 16
NW = NC * NS
CHUNK = B // NW
IDX_W = 128
IDX_ROWS = CHUNK // IDX_W
HALF = CHUNK // 2
GLOBAL_MEAN = 3.5


def _mf_body(uid_hbm, iid_hbm, w_hbm, u_hbm, bu_hbm, bi_hbm, out_hbm,
             uid_v, iid_v, wrow_v, urow_v, wpoff_v, upoff_v,
             wbuf, ubuf, bu_v, bi_v, out_v, sem):
    wid = lax.axis_index("s") * NC + lax.axis_index("c")
    row0 = wid * IDX_ROWS
    pltpu.sync_copy(uid_hbm.at[pl.ds(row0, IDX_ROWS)], uid_v)
    pltpu.sync_copy(iid_hbm.at[pl.ds(row0, IDX_ROWS)], iid_v)

    for j in range(IDX_ROWS):
        for o in range(0, IDX_W, L):
            sl = pl.ds(o, L)
            fl = pl.ds(j * IDX_W + o, L)
            u16 = uid_v[j, sl]
            i16 = iid_v[j, sl]
            wrow_v[j, sl] = lax.shift_right_logical(u16, 2)
            urow_v[j, sl] = lax.shift_right_logical(i16, 2)
            wpoff_v[fl] = lax.shift_left(u16 & 3, 5)
            upoff_v[fl] = lax.shift_left(i16 & 3, 5)

    copies = []
    for j in range(IDX_ROWS):
        sl = pl.ds(j * IDX_W, IDX_W)
        copies.append(pltpu.async_copy(bu_hbm.at[uid_v.at[j]], bu_v.at[sl], sem))
        copies.append(pltpu.async_copy(bi_hbm.at[iid_v.at[j]], bi_v.at[sl], sem))

    lane = lax.iota(jnp.int32, L)

    for h in range(2):
        hcopies = []
        for j in range(2):
            sl = pl.ds(j * IDX_W, IDX_W)
            r = h * 2 + j
            hcopies.append(pltpu.async_copy(w_hbm.at[wrow_v.at[r]], wbuf.at[sl], sem))
            hcopies.append(pltpu.async_copy(u_hbm.at[urow_v.at[r]], ubuf.at[sl], sem))
        for c in hcopies:
            c.wait()
        if h == 0:
            for c in copies:
                c.wait()

        def group(g, carry):
            o = g * L
            gi = h * HALF + o
            rid = o + lane
            acc = bu_v[pl.ds(gi, L)] + bi_v[pl.ds(gi, L)] + jnp.float32(GLOBAL_MEAN)
            wc = wpoff_v[pl.ds(gi, L)]
            uc = upoff_v[pl.ds(gi, L)]
            for k in range(K):
                acc = acc + (plsc.load_gather(wbuf, [rid, wc + k]) *
                             plsc.load_gather(ubuf, [rid, uc + k]))
            out_v[pl.ds(gi, L)] = acc
            return carry

        lax.fori_loop(0, HALF // L, group, 0)

    pltpu.sync_copy(out_v, out_hbm.at[pl.ds(wid * CHUNK, CHUNK)])


def kernel(user_ids, item_ids, W, U, bias_user, bias_item):
    uid2 = user_ids.astype(jnp.int32).reshape(NW * IDX_ROWS, IDX_W)
    iid2 = item_ids.astype(jnp.int32).reshape(NW * IDX_ROWS, IDX_W)
    w2 = W.reshape(W.shape[0] // FOLD, K * FOLD)
    u2 = U.reshape(U.shape[0] // FOLD, K * FOLD)
    mesh = plsc.VectorSubcoreMesh(core_axis_name="c", subcore_axis_name="s",
                                  num_cores=NC, num_subcores=NS)
    f = pl.kernel(
        _mf_body,
        out_type=jax.ShapeDtypeStruct((B,), jnp.float32),
        mesh=mesh,
        compiler_params=pltpu.CompilerParams(needs_layout_passes=False,
                                             use_tc_tiling_on_sc=True),
        scratch_types=[
            pltpu.VMEM((IDX_ROWS, IDX_W), jnp.int32),
            pltpu.VMEM((IDX_ROWS, IDX_W), jnp.int32),
            pltpu.VMEM((IDX_ROWS, IDX_W), jnp.int32),
            pltpu.VMEM((IDX_ROWS, IDX_W), jnp.int32),
            pltpu.VMEM((CHUNK,), jnp.int32),
            pltpu.VMEM((CHUNK,), jnp.int32),
            pltpu.VMEM((HALF, K * FOLD), jnp.float32),
            pltpu.VMEM((HALF, K * FOLD), jnp.float32),
            pltpu.VMEM((CHUNK,), jnp.float32),
            pltpu.VMEM((CHUNK,), jnp.float32),
            pltpu.VMEM((CHUNK,), jnp.float32),
            pltpu.SemaphoreType.DMA,
        ],
    )
    return f(uid2, iid2, w2, u2, bias_user, bias_item)


# speedup vs baseline: 4.7228x; 1.0098x over previous
import jax
import jax.numpy as jnp
from jax import lax
from jax.experimental import pallas as pl
from jax.experimental.pallas import tpu as pltpu
from jax.experimental.pallas import tpu_sc as plsc

B = 16384
K = 32
NC, NS, L = 2, 16, 16
NW = NC * NS
CHUNK = B // NW
IDX_W = 128
IDX_ROWS = CHUNK // IDX_W
GROUPS = CHUNK // L
GLOBAL_MEAN = 3.5


def _mf_body(uid_hbm, iid_hbm, w_hbm, u_hbm, bu_hbm, bi_hbm, out_hbm,
             uid_v, iid_v, wrows_v, urows_v, bu_v, bi_v, out_v, sem):
    wid = lax.axis_index("s") * NC + lax.axis_index("c")
    row0 = wid * IDX_ROWS
    pltpu.sync_copy(uid_hbm.at[pl.ds(row0, IDX_ROWS)], uid_v)
    pltpu.sync_copy(iid_hbm.at[pl.ds(row0, IDX_ROWS)], iid_v)

    copies = []
    for j in range(IDX_ROWS):
        sl = pl.ds(j * IDX_W, IDX_W)
        copies.append(pltpu.async_copy(w_hbm.at[uid_v.at[j]], wrows_v.at[sl], sem))
        copies.append(pltpu.async_copy(u_hbm.at[iid_v.at[j]], urows_v.at[sl], sem))
        copies.append(pltpu.async_copy(bu_hbm.at[uid_v.at[j]], bu_v.at[sl], sem))
        copies.append(pltpu.async_copy(bi_hbm.at[iid_v.at[j]], bi_v.at[sl], sem))
    for c in copies:
        c.wait()

    lane = lax.iota(jnp.int32, L)

    def group(g, carry):
        o = g * L
        rid = o + lane
        acc = bu_v[pl.ds(o, L)] + bi_v[pl.ds(o, L)] + jnp.float32(GLOBAL_MEAN)
        for k in range(K):
            kv = jnp.full((L,), k, jnp.int32)
            acc = acc + (plsc.load_gather(wrows_v, [rid, kv]) *
                         plsc.load_gather(urows_v, [rid, kv]))
        out_v[pl.ds(o, L)] = acc
        return carry

    lax.fori_loop(0, GROUPS, group, 0)
    pltpu.sync_copy(out_v, out_hbm.at[pl.ds(wid * CHUNK, CHUNK)])


def kernel(user_ids, item_ids, W, U, bias_user, bias_item):
    uid2 = user_ids.astype(jnp.int32).reshape(NW * IDX_ROWS, IDX_W)
    iid2 = item_ids.astype(jnp.int32).reshape(NW * IDX_ROWS, IDX_W)
    mesh = plsc.VectorSubcoreMesh(core_axis_name="c", subcore_axis_name="s",
                                  num_cores=NC, num_subcores=NS)
    f = pl.kernel(
        _mf_body,
        out_type=jax.ShapeDtypeStruct((B,), jnp.float32),
        mesh=mesh,
        compiler_params=pltpu.CompilerParams(needs_layout_passes=False,
                                             use_tc_tiling_on_sc=False),
        scratch_types=[
            pltpu.VMEM((IDX_ROWS, IDX_W), jnp.int32),
            pltpu.VMEM((IDX_ROWS, IDX_W), jnp.int32),
            pltpu.VMEM((CHUNK, K), jnp.float32),
            pltpu.VMEM((CHUNK, K), jnp.float32),
            pltpu.VMEM((CHUNK,), jnp.float32),
            pltpu.VMEM((CHUNK,), jnp.float32),
            pltpu.VMEM((CHUNK,), jnp.float32),
            pltpu.SemaphoreType.DMA,
        ],
    )
    return f(uid2, iid2, W, U, bias_user, bias_item)
